# Initial kernel scaffold; baseline (speedup 1.0000x reference)
import jax
import jax.numpy as jnp
from jax.experimental import pallas as pl


def kernel(x, edge_index, W_ih0, W_hh0, b_ih0, b_hh0, W_ih1, W_hh1, b_ih1, b_hh1, att_W, att_u, att_Wo, att_bo, g1_W, g1_as, g1_ad, g1_b, g3_W, g3_as, g3_ad, g3_b, lin_W, lin_b):
    raise NotImplementedError("write your pallas kernel here")


# speedup vs baseline: 10.9743x; 10.9743x over previous
import jax
import jax.numpy as jnp
from jax import lax
from jax.experimental import pallas as pl
from jax.experimental.pallas import tpu as pltpu
from jax.experimental.pallas import tpu_sc as plsc

N = 50000
T = 20
IN_DIM = 8
HID = 64
OUT_CH = 32
HCH = OUT_CH // 2
NUM_CLASSES = 3

NB = 2048
NPAD = 51200
NBLOCKS = NPAD // NB

_NC, _NS, _L = 2, 16, 16
_B = 128
_RPT = NPAD // _NS


def _lstm_step(xt, h, c, W_ih, W_hh, b):
    g = (lax.dot_general(xt, W_ih, (((1,), (1,)), ((), ())),
                         preferred_element_type=jnp.float32)
         + lax.dot_general(h, W_hh, (((1,), (1,)), ((), ())),
                           preferred_element_type=jnp.float32) + b)
    i = jax.nn.sigmoid(g[:, 0:HID])
    f = jax.nn.sigmoid(g[:, HID:2 * HID])
    gg = jnp.tanh(g[:, 2 * HID:3 * HID])
    o = jax.nn.sigmoid(g[:, 3 * HID:4 * HID])
    c = f * c + i * gg
    h = o * jnp.tanh(c)
    return h, c


def _stage_a_body(x_ref, Wih0_ref, Whh0_ref, b0_ref, Wih1_ref, Whh1_ref, b1_ref,
                  attW_ref, attu_ref, attWo_ref, attbo_ref,
                  g1W_ref, g1as_ref, g1ad_ref,
                  hf_ref, sd_ref, hs1_ref):
    Wih0 = Wih0_ref[...]
    Whh0 = Whh0_ref[...]
    b0 = b0_ref[...]
    Wih1 = Wih1_ref[...]
    Whh1 = Whh1_ref[...]
    b1 = b1_ref[...]
    attW = attW_ref[...]
    attu = attu_ref[...]

    z = jnp.zeros((NB, HID), jnp.float32)
    h0, c0, h1, c1 = z, z, z, z
    scs = []
    for t in range(T):
        xt = x_ref[:, t * IN_DIM:(t + 1) * IN_DIM]
        h0, c0 = _lstm_step(xt, h0, c0, Wih0, Whh0, b0)
        h1, c1 = _lstm_step(h0, h1, c1, Wih1, Whh1, b1)
        hs1_ref[:, t * HID:(t + 1) * HID] = h1
        sc = lax.dot_general(jnp.tanh(jnp.dot(h1, attW,
                                              preferred_element_type=jnp.float32)),
                             attu, (((1,), (0,)), ((), ())),
                             preferred_element_type=jnp.float32)
        scs.append(sc[:, None])

    scores = jnp.concatenate(scs, axis=1)
    m = jnp.max(scores, axis=1)
    a = jnp.exp(scores - m[:, None])
    a = a / jnp.sum(a, axis=1)[:, None]

    ctx = jnp.zeros((NB, HID), jnp.float32)
    for t in range(T):
        ctx = ctx + a[:, t:t + 1] * hs1_ref[:, t * HID:(t + 1) * HID]

    x1 = jnp.dot(ctx, attWo_ref[...], preferred_element_type=jnp.float32) + attbo_ref[...][None, :]
    hf = jnp.dot(x1, g1W_ref[...], preferred_element_type=jnp.float32)
    hf_ref[0] = hf[:, :HCH]
    hf_ref[1] = hf[:, HCH:]
    s = lax.dot_general(hf, g1as_ref[0], (((1,), (0,)), ((), ())),
                        preferred_element_type=jnp.float32)
    d = lax.dot_general(hf, g1ad_ref[0], (((1,), (0,)), ((), ())),
                        preferred_element_type=jnp.float32)
    sd_ref[...] = jnp.concatenate([s[:, None], d[:, None]], axis=1)


def _stage_a(xT, Wih0, Whh0, b0, Wih1, Whh1, b1, attW, attu, attWo, attbo,
             g1W, g1as, g1ad):
    full = lambda s: pl.BlockSpec(s, lambda i: (0,) * len(s))
    return pl.pallas_call(
        _stage_a_body,
        grid=(NBLOCKS,),
        in_specs=[
            pl.BlockSpec((NB, T * IN_DIM), lambda i: (i, 0)),
            full((4 * HID, IN_DIM)), full((4 * HID, HID)), full((4 * HID,)),
            full((4 * HID, HID)), full((4 * HID, HID)), full((4 * HID,)),
            full((HID, HID)), full((HID,)), full((HID, HID)), full((HID,)),
            full((HID, OUT_CH)), full((1, OUT_CH)), full((1, OUT_CH)),
        ],
        out_specs=[
            pl.BlockSpec((2, NB, HCH), lambda i: (0, i, 0)),
            pl.BlockSpec((NB, 2), lambda i: (i, 0)),
        ],
        out_shape=[
            jax.ShapeDtypeStruct((2, NPAD, HCH), jnp.float32),
            jax.ShapeDtypeStruct((NPAD, 2), jnp.float32),
        ],
        scratch_shapes=[
            pltpu.VMEM((NB, T * HID), jnp.float32),
        ],
    )(xT, Wih0, Whh0, b0, Wih1, Whh1, b1, attW, attu, attWo, attbo,
      g1W, g1as, g1ad)


def _make_gat_edges(n_blocks):
    ept = n_blocks * _B

    def body(src_hbm, dst_hbm, s_hbm, d_hbm, h_hbm, smax_hbm, znum_hbm, zden_hbm,
             num_out, den_out,
             srcv, dstv, sg, dg, hrows, whrows, wbuf, smax_v, stage_num, stage_den,
             num_acc, den_acc, sem1, sem2, sem3):
        cid = lax.axis_index("c")
        sid = lax.axis_index("s")

        rs = sid * _RPT
        pltpu.sync_copy(znum_hbm, stage_num)
        pltpu.sync_copy(stage_num, num_acc.at[pl.ds(rs, _RPT)])
        pltpu.sync_copy(zden_hbm, stage_den)
        pltpu.sync_copy(stage_den, den_acc.at[pl.ds(rs, _RPT)])
        pltpu.sync_copy(smax_hbm, smax_v)
        plsc.subcore_barrier()

        smax = smax_v[...]
        base = sid * ept

        def blk(b, carry):
            off = base + b * _B
            pltpu.sync_copy(src_hbm.at[pl.ds(off, _B)], srcv)
            pltpu.sync_copy(dst_hbm.at[pl.ds(off, _B)], dstv)
            cp1 = pltpu.async_copy(s_hbm.at[srcv], sg, sem1)
            cp2 = pltpu.async_copy(d_hbm.at[dstv], dg, sem2)
            cp3 = pltpu.async_copy(h_hbm.at[cid].at[srcv], hrows, sem3)
            cp1.wait()
            cp2.wait()
            cp3.wait()
            for k in range(_B // _L):
                sv = sg[pl.ds(k * _L, _L)]
                dv = dg[pl.ds(k * _L, _L)]
                e = sv + dv
                e = jnp.where(e >= 0, e, 0.2 * e)
                mm = smax + dv
                mm = jnp.where(mm >= 0, mm, 0.2 * mm)
                wv = jnp.exp(e - mm)
                wbuf[pl.ds(k * _L, _L)] = wv
                for j in range(_L):
                    i = k * _L + j
                    whrows[i, pl.ds(0, _L)] = hrows[i, pl.ds(0, _L)] * wv[j]
            pltpu.sync_copy(whrows, num_acc.at[dstv], add=True)

            @pl.when(cid == 1)
            def _():
                pltpu.sync_copy(wbuf, den_acc.at[dstv], add=True)

            return carry

        lax.fori_loop(0, n_blocks, blk, 0, unroll=1)

        plsc.subcore_barrier()
        pltpu.sync_copy(num_acc.at[pl.ds(rs, _RPT)], stage_num)
        pltpu.sync_copy(stage_num, num_out.at[cid, pl.ds(rs, _RPT)])

        @pl.when(cid == 1)
        def _():
            pltpu.sync_copy(den_acc.at[pl.ds(rs, _RPT)], stage_den)
            pltpu.sync_copy(stage_den, den_out.at[pl.ds(rs, _RPT)])

    kern = pl.kernel(
        body,
        out_type=(jax.ShapeDtypeStruct((_NC, NPAD, HCH), jnp.float32),
                  jax.ShapeDtypeStruct((NPAD,), jnp.float32)),
        mesh=plsc.VectorSubcoreMesh(core_axis_name="c", subcore_axis_name="s"),
        scratch_types=[
            pltpu.VMEM((_B,), jnp.int32),
            pltpu.VMEM((_B,), jnp.int32),
            pltpu.VMEM((_B,), jnp.float32),
            pltpu.VMEM((_B,), jnp.float32),
            pltpu.VMEM((_B, HCH), jnp.float32),
            pltpu.VMEM((_B, HCH), jnp.float32),
            pltpu.VMEM((_B,), jnp.float32),
            pltpu.VMEM((_L,), jnp.float32),
            pltpu.VMEM((_RPT, HCH), jnp.float32),
            pltpu.VMEM((_RPT,), jnp.float32),
            pltpu.VMEM_SHARED((NPAD, HCH), jnp.float32),
            pltpu.VMEM_SHARED((NPAD,), jnp.float32),
            pltpu.SemaphoreType.DMA,
            pltpu.SemaphoreType.DMA,
            pltpu.SemaphoreType.DMA,
        ],
        compiler_params=pltpu.CompilerParams(use_tc_tiling_on_sc=False),
    )
    return kern


_GAT_EDGES_CACHE = {}


def _gat_edges(srcp, dstp, s, d, h, smax_vec, znum, zden, n_blocks):
    if n_blocks not in _GAT_EDGES_CACHE:
        _GAT_EDGES_CACHE[n_blocks] = _make_gat_edges(n_blocks)
    return _GAT_EDGES_CACHE[n_blocks](srcp, dstp, s, d, h, smax_vec, znum, zden)


def _self_loop_combine(num_ref, den_ref, sd_ref, smax_ref, hf_ref, b_ref):
    v = sd_ref[...]
    s = v[:, 0]
    dd = v[:, 1]
    smax = smax_ref[0, 0]
    e = s + dd
    e = jnp.where(e >= 0, e, 0.2 * e)
    mm = smax + dd
    mm = jnp.where(mm >= 0, mm, 0.2 * mm)
    wself = jnp.exp(e - mm)
    hf = jnp.concatenate([hf_ref[0], hf_ref[1]], axis=1)
    num = jnp.concatenate([num_ref[0], num_ref[1]], axis=1) + wself[:, None] * hf
    den = den_ref[:, 0] + wself
    return num / (den + 1e-16)[:, None] + b_ref[...][None, :]


def _combine1_body(num_ref, den_ref, sd_ref, smax_ref, hf_ref, b_ref,
                   g3W_ref, g3as_ref, g3ad_ref, h2_ref, sd2_ref):
    x2 = _self_loop_combine(num_ref, den_ref, sd_ref, smax_ref, hf_ref, b_ref)
    h2 = jnp.dot(x2, g3W_ref[...], preferred_element_type=jnp.float32)
    h2_ref[0] = h2[:, :HCH]
    h2_ref[1] = h2[:, HCH:]
    s2 = lax.dot_general(h2, g3as_ref[0], (((1,), (0,)), ((), ())),
                         preferred_element_type=jnp.float32)
    d2 = lax.dot_general(h2, g3ad_ref[0], (((1,), (0,)), ((), ())),
                         preferred_element_type=jnp.float32)
    sd2_ref[...] = jnp.concatenate([s2[:, None], d2[:, None]], axis=1)


def _combine1(num, den, sd, smax11, hf, b, g3W, g3as, g3ad):
    full = lambda s: pl.BlockSpec(s, lambda i: (0,) * len(s))
    return pl.pallas_call(
        _combine1_body,
        grid=(NBLOCKS,),
        in_specs=[
            pl.BlockSpec((2, NB, HCH), lambda i: (0, i, 0)),
            pl.BlockSpec((NB, 1), lambda i: (i, 0)),
            pl.BlockSpec((NB, 2), lambda i: (i, 0)),
            full((1, 1)),
            pl.BlockSpec((2, NB, HCH), lambda i: (0, i, 0)),
            full((OUT_CH,)),
            full((OUT_CH, OUT_CH)), full((1, OUT_CH)), full((1, OUT_CH)),
        ],
        out_specs=[
            pl.BlockSpec((2, NB, HCH), lambda i: (0, i, 0)),
            pl.BlockSpec((NB, 2), lambda i: (i, 0)),
        ],
        out_shape=[
            jax.ShapeDtypeStruct((2, NPAD, HCH), jnp.float32),
            jax.ShapeDtypeStruct((NPAD, 2), jnp.float32),
        ],
    )(num, den, sd, smax11, hf, b, g3W, g3as, g3ad)


def _combine2_body(num_ref, den_ref, sd_ref, smax_ref, hf_ref, b_ref,
                   linW_ref, linb_ref, lsm_ref, x3_ref):
    x2 = _self_loop_combine(num_ref, den_ref, sd_ref, smax_ref, hf_ref, b_ref)
    x3 = jnp.maximum(
        jnp.dot(x2, linW_ref[...], preferred_element_type=jnp.float32)
        + linb_ref[...][None, :], 0.0)
    m = jnp.max(x3, axis=1, keepdims=True)
    sh = x3 - m
    lsm = sh - jnp.log(jnp.sum(jnp.exp(sh), axis=1, keepdims=True))
    lsm_ref[...] = lsm
    x3_ref[...] = x3


def _combine2(num, den, sd, smax11, hf, b, linW, linb):
    full = lambda s: pl.BlockSpec(s, lambda i: (0,) * len(s))
    return pl.pallas_call(
        _combine2_body,
        grid=(NBLOCKS,),
        in_specs=[
            pl.BlockSpec((2, NB, HCH), lambda i: (0, i, 0)),
            pl.BlockSpec((NB, 1), lambda i: (i, 0)),
            pl.BlockSpec((NB, 2), lambda i: (i, 0)),
            full((1, 1)),
            pl.BlockSpec((2, NB, HCH), lambda i: (0, i, 0)),
            full((OUT_CH,)),
            full((OUT_CH, NUM_CLASSES)), full((NUM_CLASSES,)),
        ],
        out_specs=[
            pl.BlockSpec((NB, NUM_CLASSES), lambda i: (i, 0)),
            pl.BlockSpec((NB, NUM_CLASSES), lambda i: (i, 0)),
        ],
        out_shape=[
            jax.ShapeDtypeStruct((NPAD, NUM_CLASSES), jnp.float32),
            jax.ShapeDtypeStruct((NPAD, NUM_CLASSES), jnp.float32),
        ],
    )(num, den, sd, smax11, hf, b, linW, linb)


def kernel(x, edge_index, W_ih0, W_hh0, b_ih0, b_hh0, W_ih1, W_hh1, b_ih1, b_hh1,
           att_W, att_u, att_Wo, att_bo, g1_W, g1_as, g1_ad, g1_b,
           g3_W, g3_as, g3_ad, g3_b, lin_W, lin_b):
    n = x.shape[0]
    xT = x.reshape(n, T * IN_DIM)
    if n < NPAD:
        xT = jnp.pad(xT, ((0, NPAD - n), (0, 0)))
    b0 = b_ih0 + b_hh0
    b1 = b_ih1 + b_hh1

    E = edge_index.shape[1]
    chunk = _NS * _B
    epad = chunk * ((E + chunk - 1) // chunk)
    npad_e = epad - E
    src = edge_index[0].astype(jnp.int32)
    dst = edge_index[1].astype(jnp.int32)
    srcp = jnp.concatenate([src, jnp.zeros((npad_e,), jnp.int32)])
    dstp = jnp.concatenate(
        [dst, n + (jnp.arange(npad_e, dtype=jnp.int32) % (NPAD - n))])
    n_blocks = epad // chunk

    znum = jnp.zeros((_RPT, HCH), jnp.float32)
    zden = jnp.zeros((_RPT,), jnp.float32)

    hf1, sd1 = _stage_a(xT, W_ih0, W_hh0, b0, W_ih1, W_hh1, b1,
                        att_W, att_u, att_Wo, att_bo, g1_W, g1_as, g1_ad)

    smax1 = jnp.max(sd1[:, 0])
    num1, den1 = _gat_edges(srcp, dstp, sd1[:, 0], sd1[:, 1], hf1,
                            jnp.full((_L,), smax1, jnp.float32),
                            znum, zden, n_blocks)

    h2, sd2 = _combine1(num1, den1.reshape(NPAD, 1), sd1,
                        smax1.reshape(1, 1), hf1, g1_b, g3_W, g3_as, g3_ad)

    smax2 = jnp.max(sd2[:, 0])
    num2, den2 = _gat_edges(srcp, dstp, sd2[:, 0], sd2[:, 1], h2,
                            jnp.full((_L,), smax2, jnp.float32),
                            znum, zden, n_blocks)

    lsm, x3 = _combine2(num2, den2.reshape(NPAD, 1), sd2,
                        smax2.reshape(1, 1), h2, g3_b, lin_W, lin_b)
    return lsm[:n], x3[:n]
